# Initial kernel scaffold; baseline (speedup 1.0000x reference)
import jax
import jax.numpy as jnp
from jax.experimental import pallas as pl


def kernel(element, table, W, b):
    raise NotImplementedError("write your pallas kernel here")


# speedup vs baseline: 18.1383x; 18.1383x over previous
import functools

import jax
import jax.numpy as jnp
from jax import lax
from jax.experimental import pallas as pl
from jax.experimental.pallas import tpu as pltpu
from jax.experimental.pallas import tpu_sc as plsc

NUM_ROWS = 1_000_000
EMB = 32
OUT_DIM = 32
BATCH = 16384
SEQ = 50
B_TOTAL = BATCH * SEQ

NC = 2
NS = 16
NW = NC * NS
B_PER_W = B_TOTAL // NW
CHUNK = 1280
N_CHUNKS = B_PER_W // CHUNK


def _make_sc_gather():
    mesh = plsc.VectorSubcoreMesh(core_axis_name="c", subcore_axis_name="s")

    @functools.partial(
        pl.kernel,
        out_type=jax.ShapeDtypeStruct((B_TOTAL, EMB), jnp.float32),
        mesh=mesh,
        scratch_types=[
            pltpu.VMEM((CHUNK,), jnp.int32),
            pltpu.VMEM((CHUNK, EMB), jnp.float32),
            pltpu.SemaphoreType.DMA,
        ],
        compiler_params=pltpu.CompilerParams(use_tc_tiling_on_sc=False),
    )
    def gather_k(idx_hbm, table_hbm, out_hbm, idx_v, rows_v, sem):
        wid = lax.axis_index("s") * NC + lax.axis_index("c")
        w_base = wid * B_PER_W

        def body(i, carry):
            base = w_base + i * CHUNK
            pltpu.sync_copy(idx_hbm.at[pl.ds(base, CHUNK)], idx_v)
            pltpu.async_copy(table_hbm.at[idx_v], rows_v, sem).wait()
            pltpu.sync_copy(rows_v, out_hbm.at[pl.ds(base, CHUNK)])
            return carry

        lax.fori_loop(0, N_CHUNKS, body, 0)

    return gather_k


_sc_gather = _make_sc_gather()

ROWS128 = B_TOTAL * EMB // 128
BLK = 2048


def _linear_relu_body(x_ref, w_ref, b_ref, o_ref):
    y = jnp.dot(x_ref[...], w_ref[...], preferred_element_type=jnp.float32)
    o_ref[...] = jnp.maximum(y + b_ref[...], 0.0)


def _tc_linear_relu(x, wbd, b4):
    return pl.pallas_call(
        _linear_relu_body,
        grid=(ROWS128 // BLK,),
        in_specs=[
            pl.BlockSpec((BLK, 128), lambda i: (i, 0)),
            pl.BlockSpec((128, 128), lambda i: (0, 0)),
            pl.BlockSpec((1, 128), lambda i: (0, 0)),
        ],
        out_specs=pl.BlockSpec((BLK, 128), lambda i: (i, 0)),
        out_shape=jax.ShapeDtypeStruct((ROWS128, 128), jnp.float32),
    )(x, wbd, b4)


def kernel(element, table, W, b):
    flat_idx = element.reshape(-1).astype(jnp.int32)
    gathered = _sc_gather(flat_idx, table)
    x = gathered.reshape(ROWS128, 128)
    wbd = jnp.kron(jnp.eye(4, dtype=W.dtype), W.T)
    b4 = jnp.tile(b, 4).reshape(1, 128)
    y = _tc_linear_relu(x, wbd, b4)
    return y.reshape(BATCH, SEQ, OUT_DIM)


# speedup vs baseline: 22.8083x; 1.2575x over previous
import functools

import jax
import jax.numpy as jnp
from jax import lax
from jax.experimental import pallas as pl
from jax.experimental.pallas import tpu as pltpu
from jax.experimental.pallas import tpu_sc as plsc

NUM_ROWS = 1_000_000
EMB = 32
OUT_DIM = 32
BATCH = 16384
SEQ = 50
B_TOTAL = BATCH * SEQ

NC = 2
NS = 16
NW = NC * NS
B_PER_W = B_TOTAL // NW
CHUNK = 1280
N_CHUNKS = B_PER_W // CHUNK


def _make_sc_gather():
    mesh = plsc.VectorSubcoreMesh(core_axis_name="c", subcore_axis_name="s")

    @functools.partial(
        pl.kernel,
        out_type=jax.ShapeDtypeStruct((B_TOTAL, EMB), jnp.float32),
        mesh=mesh,
        scratch_types=[
            pltpu.VMEM((CHUNK,), jnp.int32),
            pltpu.VMEM((CHUNK, EMB), jnp.float32),
            pltpu.SemaphoreType.DMA,
        ],
        compiler_params=pltpu.CompilerParams(use_tc_tiling_on_sc=False),
    )
    def gather_k(idx_hbm, table_hbm, out_hbm, idx_v, rows_v, sem):
        wid = lax.axis_index("s") * NC + lax.axis_index("c")
        w_base = wid * B_PER_W

        def body(i, carry):
            base = w_base + i * CHUNK
            pltpu.sync_copy(idx_hbm.at[pl.ds(base, CHUNK)], idx_v)
            pltpu.async_copy(table_hbm.at[idx_v], rows_v, sem).wait()
            pltpu.sync_copy(rows_v, out_hbm.at[pl.ds(base, CHUNK)])
            return carry

        lax.fori_loop(0, N_CHUNKS, body, 0)

    return gather_k


_sc_gather = _make_sc_gather()

BCH = 2048
Q = BCH // 4
NB = BATCH // BCH
ROWS128 = B_TOTAL * EMB // 128


def _linear_relu_body(x_ref, w_ref, b_ref, o_ref):
    x = x_ref[...]
    w = w_ref[...]
    bias = b_ref[...]
    for j in range(4):
        xj = x[:, j * EMB:(j + 1) * EMB]
        yj = lax.dot_general(w, xj, (((1,), (1,)), ((), ())),
                             preferred_element_type=jnp.float32)
        o_ref[0, :, j * Q:(j + 1) * Q] = jnp.maximum(yj + bias, 0.0)


def _tc_linear_relu(x128, w, b2d):
    return pl.pallas_call(
        _linear_relu_body,
        grid=(SEQ, NB),
        in_specs=[
            pl.BlockSpec((Q, 128), lambda s, bb: (s * NB + bb, 0)),
            pl.BlockSpec((EMB, EMB), lambda s, bb: (0, 0)),
            pl.BlockSpec((OUT_DIM, 1), lambda s, bb: (0, 0)),
        ],
        out_specs=pl.BlockSpec((1, OUT_DIM, BCH), lambda s, bb: (s, 0, bb)),
        out_shape=jax.ShapeDtypeStruct((SEQ, OUT_DIM, BATCH), jnp.float32),
    )(x128, w, b2d)


def kernel(element, table, W, b):
    idx = element.astype(jnp.int32).T
    idx = idx.reshape(SEQ, NB, 4, Q).swapaxes(2, 3).reshape(-1)
    gathered = _sc_gather(idx, table)
    x128 = gathered.reshape(ROWS128, 128)
    yT = _tc_linear_relu(x128, W, b.reshape(OUT_DIM, 1))
    return jnp.transpose(yT, (2, 0, 1))


# speedup vs baseline: 28.6681x; 1.2569x over previous
import functools

import jax
import jax.numpy as jnp
from jax import lax
from jax.experimental import pallas as pl
from jax.experimental.pallas import tpu as pltpu
from jax.experimental.pallas import tpu_sc as plsc

NUM_ROWS = 1_000_000
EMB = 32
OUT_DIM = 32
BATCH = 16384
SEQ = 50
B_TOTAL = BATCH * SEQ

NC = 2
NS = 16
NW = NC * NS
B_PER_W = B_TOTAL // NW
CHUNK = 1280
N_CHUNKS = B_PER_W // CHUNK


def _make_sc_gather():
    mesh = plsc.VectorSubcoreMesh(core_axis_name="c", subcore_axis_name="s")

    @functools.partial(
        pl.kernel,
        out_type=jax.ShapeDtypeStruct((B_TOTAL, EMB), jnp.float32),
        mesh=mesh,
        scratch_types=[
            pltpu.VMEM((CHUNK,), jnp.int32),
            pltpu.VMEM((CHUNK, EMB), jnp.float32),
            pltpu.SemaphoreType.DMA,
        ],
        compiler_params=pltpu.CompilerParams(use_tc_tiling_on_sc=False),
    )
    def gather_k(idx_hbm, table_hbm, out_hbm, idx_v, rows_v, sem):
        wid = lax.axis_index("s") * NC + lax.axis_index("c")
        w_base = wid * B_PER_W

        def body(i, carry):
            base = w_base + i * CHUNK
            pltpu.sync_copy(idx_hbm.at[pl.ds(base, CHUNK)], idx_v)
            pltpu.async_copy(table_hbm.at[idx_v], rows_v, sem).wait()
            pltpu.sync_copy(rows_v, out_hbm.at[pl.ds(base, CHUNK)])
            return carry

        lax.fori_loop(0, N_CHUNKS, body, 0)

    return gather_k


_sc_gather = _make_sc_gather()

BCH = BATCH
Q = BCH // 4
ROWS128 = B_TOTAL * EMB // 128


def _linear_relu_body(x_ref, w_ref, b_ref, o_ref):
    x = x_ref[...]
    w = w_ref[...]
    bias = b_ref[...]
    for j in range(4):
        xj = x[:, j * EMB:(j + 1) * EMB]
        yj = lax.dot_general(w, xj, (((1,), (1,)), ((), ())),
                             preferred_element_type=jnp.float32)
        o_ref[0, :, j * Q:(j + 1) * Q] = jnp.maximum(yj + bias, 0.0)


def _tc_linear_relu(x128, w, b2d):
    return pl.pallas_call(
        _linear_relu_body,
        grid=(SEQ,),
        in_specs=[
            pl.BlockSpec((Q, 128), lambda s: (s, 0)),
            pl.BlockSpec((EMB, EMB), lambda s: (0, 0)),
            pl.BlockSpec((OUT_DIM, 1), lambda s: (0, 0)),
        ],
        out_specs=pl.BlockSpec((1, OUT_DIM, BCH), lambda s: (s, 0, 0)),
        out_shape=jax.ShapeDtypeStruct((SEQ, OUT_DIM, BATCH), jnp.float32),
    )(x128, w, b2d)


def kernel(element, table, W, b):
    idx = element.astype(jnp.int32).T
    idx = idx.reshape(SEQ, 4, Q).swapaxes(1, 2).reshape(-1)
    gathered = _sc_gather(idx, table)
    x128 = gathered.reshape(ROWS128, 128)
    yT = _tc_linear_relu(x128, W, b.reshape(OUT_DIM, 1))
    return jnp.transpose(yT, (2, 0, 1))
